# baseline (speedup 1.0000x reference)
import functools

import jax
import jax.numpy as jnp
from jax import lax
from jax.experimental import pallas as pl
from jax.experimental.pallas import tpu as pltpu
from jax.experimental.pallas import tpu_sc as plsc

_NC = 2
_NS = 16
_NW = _NC * _NS
_L = 16
_NBUF = 4


def _sc_pool(x_by_worker, table, batch, hist, embed):
    bpw = batch // _NW
    assert bpw % 8 == 0 and bpw <= 128
    nvec = embed // _L
    mesh = plsc.VectorSubcoreMesh(core_axis_name="c", subcore_axis_name="s")

    @functools.partial(
        pl.kernel,
        mesh=mesh,
        compiler_params=pltpu.CompilerParams(use_tc_tiling_on_sc=False),
        out_type=jax.ShapeDtypeStruct((batch, embed), jnp.float32),
        scratch_types=(
            [pltpu.VMEM((hist, bpw), jnp.int32)]
            + [pltpu.VMEM((bpw, embed), jnp.float32) for _ in range(_NBUF)]
            + [pltpu.VMEM((bpw, embed), jnp.float32)]
            + [pltpu.SemaphoreType.DMA for _ in range(_NBUF)]
        ),
    )
    def pool(x_hbm, table_hbm, out_hbm, idx_v, *rest):
        bufs = rest[:_NBUF]
        acc = rest[_NBUF]
        sems = rest[_NBUF + 1:]
        wid = lax.axis_index("s") * _NC + lax.axis_index("c")
        base = wid * bpw

        pltpu.sync_copy(x_hbm.at[wid], idx_v)

        def start(j):
            return pltpu.async_copy(
                table_hbm.at[idx_v.at[j]], bufs[j % _NBUF], sems[j % _NBUF]
            )

        cps = {}
        for j in range(min(_NBUF, hist)):
            cps[j] = start(j)

        def accum(buf, first):
            def body(r, c):
                for p in range(nvec):
                    sl = pl.ds(p * _L, _L)
                    if first:
                        acc[r, sl] = buf[r, sl]
                    else:
                        acc[r, sl] = acc[r, sl] + buf[r, sl]
                return c
            lax.fori_loop(0, bpw, body, 0, unroll=4)

        for j in range(hist):
            cps[j].wait()
            accum(bufs[j % _NBUF], first=(j == 0))
            nj = j + _NBUF
            if nj < hist:
                cps[nj] = start(nj)

        pltpu.sync_copy(acc, out_hbm.at[pl.ds(base, bpw)])

    return pool(x_by_worker, table)


def _tc_mlp(pooled, w1, b1, w2, b2, batch, embed, ncls):
    blk = 512

    def body(p_ref, w1_ref, b1_ref, w2_ref, b2_ref, o_ref):
        h = jnp.dot(p_ref[...], w1_ref[...], preferred_element_type=jnp.float32)
        h = jnp.maximum(h + b1_ref[...], 0.0)
        o = jnp.dot(h, w2_ref[...], preferred_element_type=jnp.float32)
        o_ref[...] = o + b2_ref[...]

    return pl.pallas_call(
        body,
        grid=(batch // blk,),
        in_specs=[
            pl.BlockSpec((blk, embed), lambda i: (i, 0)),
            pl.BlockSpec((embed, embed), lambda i: (0, 0)),
            pl.BlockSpec((1, embed), lambda i: (0, 0)),
            pl.BlockSpec((embed, ncls), lambda i: (0, 0)),
            pl.BlockSpec((1, ncls), lambda i: (0, 0)),
        ],
        out_specs=pl.BlockSpec((blk, ncls), lambda i: (i, 0)),
        out_shape=jax.ShapeDtypeStruct((batch, ncls), jnp.float32),
    )(pooled, w1, b1, w2, b2)


def kernel(x, table, W1, b1, W2, b2):
    batch, hist = x.shape
    vocab, embed = table.shape
    ncls = W2.shape[1]

    bpw = batch // _NW
    xw = (
        x.astype(jnp.int32)
        .reshape(_NW, bpw, hist)
        .transpose(0, 2, 1)
    )

    pooled_sum = _sc_pool(xw, table, batch, hist, embed)

    w1s = W1 * (1.0 / hist)
    out = _tc_mlp(
        pooled_sum,
        w1s,
        b1.reshape(1, embed),
        W2,
        b2.reshape(1, ncls),
        batch,
        embed,
        ncls,
    )
    return out
